# baseline (speedup 1.0000x reference)
import functools

import jax
import jax.numpy as jnp
from jax import lax
from jax.experimental import pallas as pl
from jax.experimental.pallas import tpu as pltpu
from jax.experimental.pallas import tpu_sc as plsc

GAMMA = 0.01
ALPHA = 1.0

NWORK = 25
CHUNK = 200
NBUF = 4
ZCHUNK = 64


def _tc_body(x_ref, u_ref, zmu_ref, z_ref):
    x = x_ref[...]
    zmu = zmu_ref[...]
    diff = x - zmu
    kern = ALPHA * jnp.exp(-GAMMA * jnp.sum(diff * diff, axis=1,
                                            keepdims=True))
    u = u_ref[...]
    proj = jnp.dot(jnp.dot(diff, u, preferred_element_type=jnp.float32),
                   u.T, preferred_element_type=jnp.float32) + zmu
    z_ref[...] = proj * kern + x * (1.0 - kern)


def _compute_z(X_all, U, z_mu_local):
    n, d = X_all.shape
    k = U.shape[1]
    return pl.pallas_call(
        _tc_body,
        grid=(1,),
        in_specs=[
            pl.BlockSpec((k, d), lambda i: (0, 0)),
            pl.BlockSpec((d, k), lambda i: (0, 0)),
            pl.BlockSpec((1, d), lambda i: (0, 0)),
        ],
        out_specs=pl.BlockSpec((k, d), lambda i: (0, 0)),
        out_shape=jax.ShapeDtypeStruct((k, d), jnp.float32),
    )(X_all, U, z_mu_local)


def _sc_copy(X_all, Z):
    n, d = X_all.shape
    k = Z.shape[0]
    info = plsc.get_sparse_core_info()
    rows_per_w = n // NWORK
    nchunks = rows_per_w // CHUNK
    ngroups = nchunks // NBUF
    mesh = plsc.VectorSubcoreMesh(core_axis_name="c", subcore_axis_name="s")

    @functools.partial(
        pl.kernel, mesh=mesh,
        out_type=jax.ShapeDtypeStruct((n, d), jnp.float32),
        scratch_types=[
            pltpu.VMEM((NBUF, CHUNK, d), jnp.float32),
            pltpu.VMEM((ZCHUNK, d), jnp.float32),
            pltpu.SemaphoreType.DMA((NBUF,)),
            pltpu.SemaphoreType.DMA((NBUF,)),
            pltpu.SemaphoreType.DMA,
        ],
    )
    def sc_kernel(x_hbm, z_hbm, o_hbm, buf, zbuf, sem_in, sem_out, sem_z):
        wid = lax.axis_index("s") * info.num_cores + lax.axis_index("c")
        base = wid * rows_per_w

        def in_cp(row, b):
            return pltpu.make_async_copy(
                x_hbm.at[pl.ds(row, CHUNK)], buf.at[b], sem_in.at[b])

        def out_cp(row, b):
            return pltpu.make_async_copy(
                buf.at[b], o_hbm.at[pl.ds(row, CHUNK)], sem_out.at[b])

        @pl.when(wid < NWORK)
        def _work():
            for b in range(NBUF):
                in_cp(base + b * CHUNK, b).start()

            def group(g, carry):
                first = base + g * (NBUF * CHUNK)
                for b in range(NBUF):
                    row = first + b * CHUNK
                    in_cp(row, b).wait()
                    out_cp(row, b).start()
                for b in range(NBUF):
                    row = first + b * CHUNK
                    nxt = row + NBUF * CHUNK

                    @pl.when(g + 1 < ngroups)
                    def _():
                        out_cp(row, b).wait()
                        in_cp(nxt, b).start()
                return carry

            lax.fori_loop(0, ngroups, group, 0, unroll=False)

            last = base + (ngroups - 1) * (NBUF * CHUNK)
            for b in range(NBUF):
                out_cp(last + b * CHUNK, b).wait()

        @pl.when(wid == 0)
        def _patch():
            for j in range(k // ZCHUNK):
                cin = pltpu.make_async_copy(
                    z_hbm.at[pl.ds(j * ZCHUNK, ZCHUNK)], zbuf, sem_z)
                cin.start()
                cin.wait()
                cout = pltpu.make_async_copy(
                    zbuf, o_hbm.at[pl.ds(j * ZCHUNK, ZCHUNK)], sem_z)
                cout.start()
                cout.wait()

    return sc_kernel(X_all, Z)


def kernel(X_all, U, z_mu_local):
    Z = _compute_z(X_all, U, z_mu_local)
    return _sc_copy(X_all, Z)


# speedup vs baseline: 1.1139x; 1.1139x over previous
import functools

import jax
import jax.numpy as jnp
from jax import lax
from jax.experimental import pallas as pl
from jax.experimental.pallas import tpu as pltpu
from jax.experimental.pallas import tpu_sc as plsc

GAMMA = 0.01
ALPHA = 1.0

CHUNK = 200
NBUF = 3


def _tc_body(x_ref, u_ref, zmu_ref, z_ref):
    x = x_ref[...]
    zmu = zmu_ref[...]
    diff = x - zmu
    kern = ALPHA * jnp.exp(-GAMMA * jnp.sum(diff * diff, axis=1,
                                            keepdims=True))
    u = u_ref[...]
    proj = jnp.dot(jnp.dot(diff, u, preferred_element_type=jnp.float32),
                   u.T, preferred_element_type=jnp.float32) + zmu
    z_ref[...] = proj * kern + x * (1.0 - kern)


def _compute_z(X_all, U, z_mu_local):
    n, d = X_all.shape
    k = U.shape[1]
    return pl.pallas_call(
        _tc_body,
        grid=(1,),
        in_specs=[
            pl.BlockSpec((k, d), lambda i: (0, 0)),
            pl.BlockSpec((d, k), lambda i: (0, 0)),
            pl.BlockSpec((1, d), lambda i: (0, 0)),
        ],
        out_specs=pl.BlockSpec((k, d), lambda i: (0, 0)),
        out_shape=jax.ShapeDtypeStruct((k, d), jnp.float32),
    )(X_all, U, z_mu_local)


def _sc_copy(X_all, Z):
    n, d = X_all.shape
    k = Z.shape[0]
    info = plsc.get_sparse_core_info()
    nw = info.num_cores * info.num_subcores
    nchunks = n // CHUNK
    zfull = k // CHUNK
    zrem = k - zfull * CHUNK
    mesh = plsc.VectorSubcoreMesh(core_axis_name="c", subcore_axis_name="s")

    @functools.partial(
        pl.kernel, mesh=mesh,
        out_type=jax.ShapeDtypeStruct((n, d), jnp.float32),
        scratch_types=[
            pltpu.VMEM((NBUF, CHUNK, d), jnp.float32),
            pltpu.SemaphoreType.DMA((NBUF,)),
            pltpu.SemaphoreType.DMA((NBUF,)),
        ],
    )
    def sc_kernel(x_hbm, z_hbm, o_hbm, buf, sem_in, sem_out):
        wid = lax.axis_index("s") * info.num_cores + lax.axis_index("c")
        jcount = (nchunks - wid + nw - 1) // nw

        def row_of(j):
            return (wid + j * nw) * CHUNK

        def in_cp(j, b):
            return pltpu.make_async_copy(
                x_hbm.at[pl.ds(row_of(j), CHUNK)], buf.at[b], sem_in.at[b])

        def out_cp(j, b):
            return pltpu.make_async_copy(
                buf.at[b], o_hbm.at[pl.ds(row_of(j), CHUNK)], sem_out.at[b])

        def bslot(j):
            return lax.rem(j, NBUF)

        @pl.when(wid < zfull)
        def _first_z():
            pltpu.make_async_copy(
                z_hbm.at[pl.ds(wid * CHUNK, CHUNK)], buf.at[0],
                sem_in.at[0]).start()

        @pl.when(wid == zfull)
        def _first_zx():
            pltpu.make_async_copy(
                z_hbm.at[pl.ds(zfull * CHUNK, zrem)],
                buf.at[0, pl.ds(0, zrem)], sem_in.at[0]).start()
            pltpu.make_async_copy(
                x_hbm.at[pl.ds(k, CHUNK - zrem)],
                buf.at[0, pl.ds(zrem, CHUNK - zrem)], sem_in.at[0]).start()

        @pl.when(wid > zfull)
        def _first_x():
            in_cp(0, 0).start()

        @pl.when(jcount > 1)
        def _second():
            in_cp(1, 1).start()

        def body(j, carry):
            nxt = j + NBUF - 1

            @pl.when(nxt < jcount)
            def _():
                @pl.when(j >= 1)
                def _():
                    out_cp(j - 1, bslot(j - 1)).wait()

                in_cp(nxt, bslot(nxt)).start()

            in_cp(j, bslot(j)).wait()
            out_cp(j, bslot(j)).start()
            return carry

        lax.fori_loop(0, jcount, body, 0, unroll=False)

        for t in range(NBUF):
            tail = jcount - NBUF + t

            @pl.when(tail >= 0)
            def _():
                out_cp(tail, bslot(tail)).wait()

    return sc_kernel(X_all, Z)


def kernel(X_all, U, z_mu_local):
    Z = _compute_z(X_all, U, z_mu_local)
    return _sc_copy(X_all, Z)


# speedup vs baseline: 1.1144x; 1.0005x over previous
import functools

import jax
import jax.numpy as jnp
from jax import lax
from jax.experimental import pallas as pl
from jax.experimental.pallas import tpu as pltpu
from jax.experimental.pallas import tpu_sc as plsc

GAMMA = 0.01
ALPHA = 1.0

CHUNK = 200
NBUF = 4


def _tc_body(x_ref, u_ref, zmu_ref, z_ref):
    x = x_ref[...]
    zmu = zmu_ref[...]
    diff = x - zmu
    kern = ALPHA * jnp.exp(-GAMMA * jnp.sum(diff * diff, axis=1,
                                            keepdims=True))
    u = u_ref[...]
    proj = jnp.dot(jnp.dot(diff, u, preferred_element_type=jnp.float32),
                   u.T, preferred_element_type=jnp.float32) + zmu
    z_ref[...] = proj * kern + x * (1.0 - kern)


def _compute_z(X_all, U, z_mu_local):
    n, d = X_all.shape
    k = U.shape[1]
    return pl.pallas_call(
        _tc_body,
        grid=(1,),
        in_specs=[
            pl.BlockSpec((k, d), lambda i: (0, 0)),
            pl.BlockSpec((d, k), lambda i: (0, 0)),
            pl.BlockSpec((1, d), lambda i: (0, 0)),
        ],
        out_specs=pl.BlockSpec((k, d), lambda i: (0, 0)),
        out_shape=jax.ShapeDtypeStruct((k, d), jnp.float32),
    )(X_all, U, z_mu_local)


def _sc_copy(X_all, Z):
    n, d = X_all.shape
    k = Z.shape[0]
    info = plsc.get_sparse_core_info()
    nw = info.num_cores * info.num_subcores
    nchunks = n // CHUNK
    zfull = k // CHUNK
    zrem = k - zfull * CHUNK
    mesh = plsc.VectorSubcoreMesh(core_axis_name="c", subcore_axis_name="s")

    @functools.partial(
        pl.kernel, mesh=mesh,
        out_type=jax.ShapeDtypeStruct((n, d), jnp.float32),
        scratch_types=[
            pltpu.VMEM((NBUF, CHUNK, d), jnp.float32),
            pltpu.SemaphoreType.DMA((NBUF,)),
            pltpu.SemaphoreType.DMA((NBUF,)),
        ],
    )
    def sc_kernel(x_hbm, z_hbm, o_hbm, buf, sem_in, sem_out):
        wid = lax.axis_index("s") * info.num_cores + lax.axis_index("c")
        jcount = (nchunks - wid + nw - 1) // nw

        def row_of(j):
            return (wid + j * nw) * CHUNK

        def in_cp(j, b):
            return pltpu.make_async_copy(
                x_hbm.at[pl.ds(row_of(j), CHUNK)], buf.at[b], sem_in.at[b])

        def out_cp(j, b):
            return pltpu.make_async_copy(
                buf.at[b], o_hbm.at[pl.ds(row_of(j), CHUNK)], sem_out.at[b])

        def bslot(j):
            return lax.rem(j, NBUF)

        @pl.when(wid < zfull)
        def _first_z():
            pltpu.make_async_copy(
                z_hbm.at[pl.ds(wid * CHUNK, CHUNK)], buf.at[0],
                sem_in.at[0]).start()

        @pl.when(wid == zfull)
        def _first_zx():
            pltpu.make_async_copy(
                z_hbm.at[pl.ds(zfull * CHUNK, zrem)],
                buf.at[0, pl.ds(0, zrem)], sem_in.at[0]).start()
            pltpu.make_async_copy(
                x_hbm.at[pl.ds(k, CHUNK - zrem)],
                buf.at[0, pl.ds(zrem, CHUNK - zrem)], sem_in.at[0]).start()

        @pl.when(wid > zfull)
        def _first_x():
            in_cp(0, 0).start()

        for jj in range(1, NBUF - 1):
            @pl.when(jcount > jj)
            def _prime(jj=jj):
                in_cp(jj, jj).start()

        def body(j, carry):
            nxt = j + NBUF - 1

            @pl.when(nxt < jcount)
            def _():
                @pl.when(j >= 1)
                def _():
                    out_cp(j - 1, bslot(j - 1)).wait()

                in_cp(nxt, bslot(nxt)).start()

            in_cp(j, bslot(j)).wait()
            out_cp(j, bslot(j)).start()
            return carry

        lax.fori_loop(0, jcount, body, 0, unroll=False)

        for t in range(NBUF):
            tail = jcount - NBUF + t

            @pl.when(tail >= 0)
            def _():
                out_cp(tail, bslot(tail)).wait()

    return sc_kernel(X_all, Z)


def kernel(X_all, U, z_mu_local):
    Z = _compute_z(X_all, U, z_mu_local)
    return _sc_copy(X_all, Z)
